# Initial kernel scaffold; baseline (speedup 1.0000x reference)
import functools

import jax
import jax.numpy as jnp
from jax import lax
from jax.experimental import pallas as pl
from jax.experimental.pallas import tpu as pltpu
from jax.experimental.pallas import tpu_sc as plsc

N = 10000
D = 256
H = 128
E = 160000
E_PAD = 163840
CH = 128
CHUNKS = E_PAD // (16 * CH)
PER_TILE = E_PAD // 16
ACC_ROWS = N + 16
ZROWS_PER_TILE = ACC_ROWS // 16
OUT_ROWS_PER_TILE = N // 16
G = 64
R = 400
GRID = N // R

_f32 = jnp.float32


_sc_mesh = plsc.VectorSubcoreMesh(
    core_axis_name="c", subcore_axis_name="s", num_cores=2, num_subcores=16)


def _sc_agg_body(h0, h1, srcp, dstp, out0, out1, sidx, didx, rows, acc, sem):
    c = lax.axis_index("c")
    s = lax.axis_index("s")

    @pl.loop(0, 128)
    def _zr(i):
        @pl.loop(0, 8)
        def _zc(j):
            rows[i, pl.ds(j * 16, 16)] = jnp.zeros((16,), _f32)

    zbase = s * ZROWS_PER_TILE
    pltpu.sync_copy(rows.at[pl.ds(0, 128)], acc.at[pl.ds(zbase, 128)])
    pltpu.sync_copy(rows.at[pl.ds(0, 128)], acc.at[pl.ds(zbase + 128, 128)])
    pltpu.sync_copy(rows.at[pl.ds(0, 128)], acc.at[pl.ds(zbase + 256, 128)])
    pltpu.sync_copy(rows.at[pl.ds(0, 128)], acc.at[pl.ds(zbase + 384, 128)])
    pltpu.sync_copy(rows.at[pl.ds(0, 114)],
                    acc.at[pl.ds(zbase + 512, ZROWS_PER_TILE - 512)])
    plsc.subcore_barrier()

    ebase = s * PER_TILE

    def _run(table):
        @pl.loop(0, CHUNKS)
        def _chunk(i):
            off = ebase + i * CH
            pltpu.sync_copy(srcp.at[pl.ds(off, CH)], sidx)
            pltpu.sync_copy(dstp.at[pl.ds(off, CH)], didx)
            pltpu.async_copy(table.at[sidx], rows, sem).wait()
            pltpu.sync_copy(rows, acc.at[didx], add=True)

    @pl.when(c == 0)
    def _():
        _run(h0)

    @pl.when(c == 1)
    def _():
        _run(h1)

    plsc.subcore_barrier()
    rbase = s * OUT_ROWS_PER_TILE

    @pl.when(c == 0)
    def _():
        pltpu.sync_copy(acc.at[pl.ds(rbase, OUT_ROWS_PER_TILE)],
                        out0.at[pl.ds(rbase, OUT_ROWS_PER_TILE)])

    @pl.when(c == 1)
    def _():
        pltpu.sync_copy(acc.at[pl.ds(rbase, OUT_ROWS_PER_TILE)],
                        out1.at[pl.ds(rbase, OUT_ROWS_PER_TILE)])


_sc_agg = pl.kernel(
    _sc_agg_body,
    out_type=[jax.ShapeDtypeStruct((N, H), _f32),
              jax.ShapeDtypeStruct((N, H), _f32)],
    mesh=_sc_mesh,
    scratch_types=[
        pltpu.VMEM((CH,), jnp.int32),
        pltpu.VMEM((CH,), jnp.int32),
        pltpu.VMEM((CH, H), _f32),
        pltpu.VMEM_SHARED((ACC_ROWS, H), _f32),
        pltpu.SemaphoreType.DMA,
    ],
)


def _gelu(y):
    return 0.5 * y * (1.0 + lax.erf(y * 0.7071067811865476))


def _layer_a_body(h0_ref, h1_ref, a0_ref, a1_ref, wr_ref, br_ref, w1_ref,
                  b1_ref, w2_ref, b2_ref, eps_ref, h2_ref, id_ref, st_ref):
    i = pl.program_id(0)
    e1 = 1.0 + eps_ref[0, 0]
    h0 = h0_ref[...]
    h1 = h1_ref[...]
    u0 = e1 * h0 + a0_ref[...]
    u1 = e1 * h1 + a1_ref[...]
    ident = (jnp.dot(h0, wr_ref[0:H, :], preferred_element_type=_f32)
             + jnp.dot(h1, wr_ref[H:D, :], preferred_element_type=_f32)
             + br_ref[...])
    t = (jnp.dot(u0, w1_ref[0:H, :], preferred_element_type=_f32)
         + jnp.dot(u1, w1_ref[H:D, :], preferred_element_type=_f32)
         + b1_ref[...])
    t = jnp.maximum(t, 0.0)
    h2 = jnp.dot(t, w2_ref[...], preferred_element_type=_f32) + b2_ref[...]
    h2_ref[...] = h2
    id_ref[...] = ident

    @pl.when(i == 0)
    def _():
        st_ref[...] = jnp.zeros((2, D), _f32)

    st_ref[0:1, :] += jnp.sum(h2, axis=0, keepdims=True)
    st_ref[1:2, :] += jnp.sum(h2 * h2, axis=0, keepdims=True)


def _layer_a(h0, h1, a0, a1, wr, br, w1, b1, w2, b2, eps):
    blk = lambda w: pl.BlockSpec((R, w), lambda i: (i, 0))
    full = lambda a, b: pl.BlockSpec((a, b), lambda i: (0, 0))
    return pl.pallas_call(
        _layer_a_body,
        grid=(GRID,),
        in_specs=[blk(H), blk(H), blk(H), blk(H),
                  full(D, D), full(1, D), full(D, D), full(1, D),
                  full(D, D), full(1, D), full(1, 1)],
        out_specs=[blk(D), blk(D), full(2, D)],
        out_shape=[jax.ShapeDtypeStruct((N, D), _f32),
                   jax.ShapeDtypeStruct((N, D), _f32),
                   jax.ShapeDtypeStruct((2, D), _f32)],
    )(h0, h1, a0, a1, wr, br, w1, b1, w2, b2, eps)


def _layer_b_body(h2_ref, id_ref, st_ref, g_ref, be_ref, o0_ref, o1_ref):
    st = st_ref[...]
    mu = st[0:1, :] * (1.0 / N)
    var = st[1:2, :] * (1.0 / N) - mu * mu
    inv = lax.rsqrt(var + 1e-5) * g_ref[...]
    y = (h2_ref[...] - mu) * inv + be_ref[...] + id_ref[...]
    y = _gelu(y)
    o0_ref[...] = y[:, 0:H]
    o1_ref[...] = y[:, H:D]


def _layer_b(h2, ident, st, gamma, beta):
    blk = lambda w: pl.BlockSpec((R, w), lambda i: (i, 0))
    full = lambda a, b: pl.BlockSpec((a, b), lambda i: (0, 0))
    return pl.pallas_call(
        _layer_b_body,
        grid=(GRID,),
        in_specs=[blk(D), blk(D), full(2, D), full(1, D), full(1, D)],
        out_specs=[blk(H), blk(H)],
        out_shape=[jax.ShapeDtypeStruct((N, H), _f32),
                   jax.ShapeDtypeStruct((N, H), _f32)],
    )(h2, ident, st, gamma, beta)


def _head_body(h0_ref, h1_ref, b_ref, wf1_ref, bf1_ref, wf2_ref, bf2_ref,
               out_ref, sums, cntm):
    i = pl.program_id(0)

    @pl.when(i == 0)
    def _():
        sums[...] = jnp.zeros((G, D), _f32)
        cntm[...] = jnp.zeros((G, H), _f32)

    oh = (b_ref[...] == lax.broadcasted_iota(jnp.int32, (R, G), 1)).astype(_f32)
    dn = (((0,), (0,)), ((), ()))
    sums[:, 0:H] += lax.dot_general(oh, h0_ref[...], dn,
                                    preferred_element_type=_f32)
    sums[:, H:D] += lax.dot_general(oh, h1_ref[...], dn,
                                    preferred_element_type=_f32)
    cntm[...] += lax.dot_general(oh, jnp.ones((R, H), _f32), dn,
                                 preferred_element_type=_f32)

    @pl.when(i == GRID - 1)
    def _():
        cnt = jnp.maximum(cntm[...], 1.0)
        p0 = sums[:, 0:H] / cnt
        p1 = sums[:, H:D] / cnt
        z = (jnp.dot(p0, wf1_ref[0:H, :], preferred_element_type=_f32)
             + jnp.dot(p1, wf1_ref[H:D, :], preferred_element_type=_f32)
             + bf1_ref[...])
        z = _gelu(z)
        out_ref[...] = (jnp.dot(z, wf2_ref[...], preferred_element_type=_f32)
                        + bf2_ref[...])


def _head(h0, h1, batch2, wf1, bf1, wf2, bf2):
    blk = lambda w: pl.BlockSpec((R, w), lambda i: (i, 0))
    full = lambda a, b: pl.BlockSpec((a, b), lambda i: (0, 0))
    return pl.pallas_call(
        _head_body,
        grid=(GRID,),
        in_specs=[blk(H), blk(H), blk(1),
                  full(D, D), full(1, D), full(D, 10), full(1, 10)],
        out_specs=pl.BlockSpec((G, 10), lambda i: (0, 0)),
        out_shape=jax.ShapeDtypeStruct((G, 10), _f32),
        scratch_shapes=[pltpu.VMEM((G, D), _f32), pltpu.VMEM((G, H), _f32)],
    )(h0, h1, batch2, wf1, bf1, wf2, bf2)


def kernel(x, edge_index, batch, params):
    src = edge_index[0]
    dst = edge_index[1]
    npad = E_PAD - E
    ar = jnp.arange(npad, dtype=jnp.int32)
    srcp = jnp.concatenate([src, (ar * 997) % N])
    dstp = jnp.concatenate([dst, N + (ar % 16)])
    batch2 = batch.reshape(N, 1)

    h0 = x[:, 0:H]
    h1 = x[:, H:D]
    for l in range(4):
        g = params[f"gin{l}"]
        bn = params[f"bn{l}"]
        rs = params[f"res{l}"]
        a0, a1 = _sc_agg(h0, h1, srcp, dstp)
        h2, ident, st = _layer_a(
            h0, h1, a0, a1, rs["W"], rs["b"].reshape(1, D),
            g["lin1"]["W"], g["lin1"]["b"].reshape(1, D),
            g["lin2"]["W"], g["lin2"]["b"].reshape(1, D),
            g["eps"].reshape(1, 1))
        h0, h1 = _layer_b(h2, ident, st, bn["gamma"].reshape(1, D),
                          bn["beta"].reshape(1, D))
    return _head(h0, h1, batch2, params["fc1"]["W"],
                 params["fc1"]["b"].reshape(1, D), params["fc2"]["W"],
                 params["fc2"]["b"].reshape(1, 10))


# speedup vs baseline: 3.5947x; 3.5947x over previous
import functools

import jax
import jax.numpy as jnp
from jax import lax
from jax.experimental import pallas as pl
from jax.experimental.pallas import tpu as pltpu
from jax.experimental.pallas import tpu_sc as plsc

N = 10000
D = 256
H = 128
E = 160000
E_PAD = 163840
CH = 128
CHUNKS = E_PAD // (16 * CH)
PER_TILE = E_PAD // 16
ACC_ROWS = 10112
ZROWS_PER_TILE = ACC_ROWS // 16
OUT_ROWS_PER_TILE = 632
OUT_TAIL = N - 15 * OUT_ROWS_PER_TILE
G = 64
R = 400
GRID = N // R

_f32 = jnp.float32


def _sc_agg_body(h0, h1, srcp, dstp, out0, out1, sidx, didx, rows, acc, sem):
    c = lax.axis_index("c")
    s = lax.axis_index("s")

    @pl.loop(0, 128)
    def _zr(i):
        @pl.loop(0, 8)
        def _zc(j):
            rows[i, pl.ds(j * 16, 16)] = jnp.zeros((16,), _f32)

    zbase = s * ZROWS_PER_TILE
    pltpu.sync_copy(rows.at[pl.ds(0, 128)], acc.at[pl.ds(zbase, 128)])
    pltpu.sync_copy(rows.at[pl.ds(0, 128)], acc.at[pl.ds(zbase + 128, 128)])
    pltpu.sync_copy(rows.at[pl.ds(0, 128)], acc.at[pl.ds(zbase + 256, 128)])
    pltpu.sync_copy(rows.at[pl.ds(0, 128)], acc.at[pl.ds(zbase + 384, 128)])
    pltpu.sync_copy(rows.at[pl.ds(0, 120)],
                    acc.at[pl.ds(zbase + 512, ZROWS_PER_TILE - 512)])
    plsc.subcore_barrier()

    ebase = s * PER_TILE

    def _run(table):
        @pl.loop(0, CHUNKS)
        def _chunk(i):
            off = ebase + i * CH
            pltpu.sync_copy(srcp.at[pl.ds(off, CH)], sidx)
            pltpu.sync_copy(dstp.at[pl.ds(off, CH)], didx)
            pltpu.async_copy(table.at[sidx], rows, sem).wait()
            pltpu.sync_copy(rows, acc.at[didx], add=True)

    @pl.when(c == 0)
    def _():
        _run(h0)

    @pl.when(c == 1)
    def _():
        _run(h1)

    plsc.subcore_barrier()
    rbase = s * OUT_ROWS_PER_TILE

    def _copy_out(out):
        @pl.when(s < 15)
        def _():
            pltpu.sync_copy(acc.at[pl.ds(rbase, OUT_ROWS_PER_TILE)],
                            out.at[pl.ds(rbase, OUT_ROWS_PER_TILE)])

        @pl.when(s == 15)
        def _():
            pltpu.sync_copy(acc.at[pl.ds(15 * OUT_ROWS_PER_TILE, OUT_TAIL)],
                            out.at[pl.ds(15 * OUT_ROWS_PER_TILE, OUT_TAIL)])

    @pl.when(c == 0)
    def _():
        _copy_out(out0)

    @pl.when(c == 1)
    def _():
        _copy_out(out1)


@functools.lru_cache(maxsize=None)
def _get_sc_agg():
    mesh = plsc.VectorSubcoreMesh(
        core_axis_name="c", subcore_axis_name="s",
        num_cores=2, num_subcores=16)
    return pl.kernel(
        _sc_agg_body,
        out_type=[jax.ShapeDtypeStruct((N, H), _f32),
                  jax.ShapeDtypeStruct((N, H), _f32)],
        mesh=mesh,
        scratch_types=[
            pltpu.VMEM((CH,), jnp.int32),
            pltpu.VMEM((CH,), jnp.int32),
            pltpu.VMEM((CH, H), _f32),
            pltpu.VMEM_SHARED((ACC_ROWS, H), _f32),
            pltpu.SemaphoreType.DMA,
        ],
    )


def _gelu(y):
    return 0.5 * y * (1.0 + lax.erf(y * 0.7071067811865476))


def _layer_a_body(h0_ref, h1_ref, a0_ref, a1_ref, wr_ref, br_ref, w1_ref,
                  b1_ref, w2_ref, b2_ref, eps_ref, h2_ref, id_ref, st_ref):
    i = pl.program_id(0)
    e1 = 1.0 + eps_ref[0, 0]
    h0 = h0_ref[...]
    h1 = h1_ref[...]
    u0 = e1 * h0 + a0_ref[...]
    u1 = e1 * h1 + a1_ref[...]
    ident = (jnp.dot(h0, wr_ref[0:H, :], preferred_element_type=_f32)
             + jnp.dot(h1, wr_ref[H:D, :], preferred_element_type=_f32)
             + br_ref[...])
    t = (jnp.dot(u0, w1_ref[0:H, :], preferred_element_type=_f32)
         + jnp.dot(u1, w1_ref[H:D, :], preferred_element_type=_f32)
         + b1_ref[...])
    t = jnp.maximum(t, 0.0)
    h2 = jnp.dot(t, w2_ref[...], preferred_element_type=_f32) + b2_ref[...]
    h2_ref[...] = h2
    id_ref[...] = ident

    @pl.when(i == 0)
    def _():
        st_ref[...] = jnp.zeros((2, D), _f32)

    st_ref[0:1, :] += jnp.sum(h2, axis=0, keepdims=True)
    st_ref[1:2, :] += jnp.sum(h2 * h2, axis=0, keepdims=True)


def _layer_a(h0, h1, a0, a1, wr, br, w1, b1, w2, b2, eps):
    blk = lambda w: pl.BlockSpec((R, w), lambda i: (i, 0))
    full = lambda a, b: pl.BlockSpec((a, b), lambda i: (0, 0))
    return pl.pallas_call(
        _layer_a_body,
        grid=(GRID,),
        in_specs=[blk(H), blk(H), blk(H), blk(H),
                  full(D, D), full(1, D), full(D, D), full(1, D),
                  full(D, D), full(1, D), full(1, 1)],
        out_specs=[blk(D), blk(D), full(2, D)],
        out_shape=[jax.ShapeDtypeStruct((N, D), _f32),
                   jax.ShapeDtypeStruct((N, D), _f32),
                   jax.ShapeDtypeStruct((2, D), _f32)],
    )(h0, h1, a0, a1, wr, br, w1, b1, w2, b2, eps)


def _layer_b_body(h2_ref, id_ref, st_ref, g_ref, be_ref, o0_ref, o1_ref):
    st = st_ref[...]
    mu = st[0:1, :] * (1.0 / N)
    var = st[1:2, :] * (1.0 / N) - mu * mu
    inv = lax.rsqrt(var + 1e-5) * g_ref[...]
    y = (h2_ref[...] - mu) * inv + be_ref[...] + id_ref[...]
    y = _gelu(y)
    o0_ref[...] = y[:, 0:H]
    o1_ref[...] = y[:, H:D]


def _layer_b(h2, ident, st, gamma, beta):
    blk = lambda w: pl.BlockSpec((R, w), lambda i: (i, 0))
    full = lambda a, b: pl.BlockSpec((a, b), lambda i: (0, 0))
    return pl.pallas_call(
        _layer_b_body,
        grid=(GRID,),
        in_specs=[blk(D), blk(D), full(2, D), full(1, D), full(1, D)],
        out_specs=[blk(H), blk(H)],
        out_shape=[jax.ShapeDtypeStruct((N, H), _f32),
                   jax.ShapeDtypeStruct((N, H), _f32)],
    )(h2, ident, st, gamma, beta)


def _head_body(h0_ref, h1_ref, b_ref, wf1_ref, bf1_ref, wf2_ref, bf2_ref,
               out_ref, sums, cntm):
    i = pl.program_id(0)

    @pl.when(i == 0)
    def _():
        sums[...] = jnp.zeros((G, D), _f32)
        cntm[...] = jnp.zeros((G, H), _f32)

    oh = (b_ref[...] == lax.broadcasted_iota(jnp.int32, (R, G), 1)).astype(_f32)
    dn = (((0,), (0,)), ((), ()))
    sums[:, 0:H] += lax.dot_general(oh, h0_ref[...], dn,
                                    preferred_element_type=_f32)
    sums[:, H:D] += lax.dot_general(oh, h1_ref[...], dn,
                                    preferred_element_type=_f32)
    cntm[...] += lax.dot_general(oh, jnp.ones((R, H), _f32), dn,
                                 preferred_element_type=_f32)

    @pl.when(i == GRID - 1)
    def _():
        cnt = jnp.maximum(cntm[...], 1.0)
        p0 = sums[:, 0:H] / cnt
        p1 = sums[:, H:D] / cnt
        z = (jnp.dot(p0, wf1_ref[0:H, :], preferred_element_type=_f32)
             + jnp.dot(p1, wf1_ref[H:D, :], preferred_element_type=_f32)
             + bf1_ref[...])
        z = _gelu(z)
        out_ref[...] = (jnp.dot(z, wf2_ref[...], preferred_element_type=_f32)
                        + bf2_ref[...])


def _head(h0, h1, batch2, wf1, bf1, wf2, bf2):
    blk = lambda w: pl.BlockSpec((R, w), lambda i: (i, 0))
    full = lambda a, b: pl.BlockSpec((a, b), lambda i: (0, 0))
    return pl.pallas_call(
        _head_body,
        grid=(GRID,),
        in_specs=[blk(H), blk(H), blk(1),
                  full(D, D), full(1, D), full(D, 10), full(1, 10)],
        out_specs=pl.BlockSpec((G, 10), lambda i: (0, 0)),
        out_shape=jax.ShapeDtypeStruct((G, 10), _f32),
        scratch_shapes=[pltpu.VMEM((G, D), _f32), pltpu.VMEM((G, H), _f32)],
    )(h0, h1, batch2, wf1, bf1, wf2, bf2)


def kernel(x, edge_index, batch, params):
    src = edge_index[0]
    dst = edge_index[1]
    npad = E_PAD - E
    ar = jnp.arange(npad, dtype=jnp.int32)
    srcp = jnp.concatenate([src, (ar * 997) % N])
    dstp = jnp.concatenate([dst, N + (ar % 16)])
    batch2 = batch.reshape(N, 1)

    h0 = x[:, 0:H]
    h1 = x[:, H:D]
    for l in range(4):
        g = params[f"gin{l}"]
        bn = params[f"bn{l}"]
        rs = params[f"res{l}"]
        a0, a1 = _get_sc_agg()(h0, h1, srcp, dstp)
        h2, ident, st = _layer_a(
            h0, h1, a0, a1, rs["W"], rs["b"].reshape(1, D),
            g["lin1"]["W"], g["lin1"]["b"].reshape(1, D),
            g["lin2"]["W"], g["lin2"]["b"].reshape(1, D),
            g["eps"].reshape(1, 1))
        h0, h1 = _layer_b(h2, ident, st, bn["gamma"].reshape(1, D),
                          bn["beta"].reshape(1, D))
    return _head(h0, h1, batch2, params["fc1"]["W"],
                 params["fc1"]["b"].reshape(1, D), params["fc2"]["W"],
                 params["fc2"]["b"].reshape(1, 10))


# speedup vs baseline: 6.0299x; 1.6774x over previous
import functools

import jax
import jax.numpy as jnp
from jax import lax
from jax.experimental import pallas as pl
from jax.experimental.pallas import tpu as pltpu
from jax.experimental.pallas import tpu_sc as plsc

N = 10000
D = 256
H = 128
E = 160000
E_PAD = 163840
CH = 128
CHUNKS = E_PAD // (16 * CH)
PER_TILE = E_PAD // 16
ACC_ROWS = 10112
ZROWS_PER_TILE = ACC_ROWS // 16
OUT_ROWS_PER_TILE = 632
OUT_TAIL = N - 15 * OUT_ROWS_PER_TILE
G = 64
R = 400
GRID = N // R

_f32 = jnp.float32


def _sc_agg_body(h0, h1, srcp, dstp, out0, out1,
                 sidx0, didx0, sidx1, didx1, rows0, rows1, acc,
                 semi0, semi1, semg0, semg1):
    c = lax.axis_index("c")
    s = lax.axis_index("s")

    @pl.loop(0, CH)
    def _zr(i):
        @pl.loop(0, 8)
        def _zc(j):
            rows0[i, pl.ds(j * 16, 16)] = jnp.zeros((16,), _f32)

    zbase = s * ZROWS_PER_TILE
    for k in range(4):
        pltpu.sync_copy(rows0.at[pl.ds(0, CH)],
                        acc.at[pl.ds(zbase + CH * k, CH)])
    pltpu.sync_copy(rows0.at[pl.ds(0, ZROWS_PER_TILE - 4 * CH)],
                    acc.at[pl.ds(zbase + 4 * CH, ZROWS_PER_TILE - 4 * CH)])
    plsc.subcore_barrier()

    ebase = s * PER_TILE
    bufs = ((sidx0, didx0, rows0, semi0, semg0),
            (sidx1, didx1, rows1, semi1, semg1))

    def _idx_start(i, sb, db, semi):
        off = ebase + i * CH
        pltpu.async_copy(srcp.at[pl.ds(off, CH)], sb, semi)
        pltpu.async_copy(dstp.at[pl.ds(off, CH)], db, semi)

    def _idx_wait(i, sb, db, semi):
        off = ebase + i * CH
        pltpu.make_async_copy(srcp.at[pl.ds(off, CH)], sb, semi).wait()
        pltpu.make_async_copy(dstp.at[pl.ds(off, CH)], db, semi).wait()

    def _run(table):
        _idx_start(0, sidx0, didx0, semi0)
        _idx_start(1, sidx1, didx1, semi1)
        _idx_wait(0, sidx0, didx0, semi0)
        pltpu.async_copy(table.at[sidx0], rows0, semg0)

        @pl.loop(0, CHUNKS // 2)
        def _pair(g):
            for b in range(2):
                sb, db, rb, semi, semg = bufs[b]
                nsb, ndb, nrb, nsemi, nsemg = bufs[1 - b]
                i = 2 * g + b

                @pl.when(i + 1 < CHUNKS)
                def _():
                    _idx_wait(i + 1, nsb, ndb, nsemi)
                    pltpu.async_copy(table.at[nsb], nrb, nsemg)

                pltpu.make_async_copy(table.at[sb], rb, semg).wait()
                pltpu.sync_copy(rb, acc.at[db], add=True)

                @pl.when(i + 2 < CHUNKS)
                def _():
                    _idx_start(i + 2, sb, db, semi)

    @pl.when(c == 0)
    def _():
        _run(h0)

    @pl.when(c == 1)
    def _():
        _run(h1)

    plsc.subcore_barrier()
    rbase = s * OUT_ROWS_PER_TILE

    def _copy_out(out):
        @pl.when(s < 15)
        def _():
            pltpu.sync_copy(acc.at[pl.ds(rbase, OUT_ROWS_PER_TILE)],
                            out.at[pl.ds(rbase, OUT_ROWS_PER_TILE)])

        @pl.when(s == 15)
        def _():
            pltpu.sync_copy(acc.at[pl.ds(15 * OUT_ROWS_PER_TILE, OUT_TAIL)],
                            out.at[pl.ds(15 * OUT_ROWS_PER_TILE, OUT_TAIL)])

    @pl.when(c == 0)
    def _():
        _copy_out(out0)

    @pl.when(c == 1)
    def _():
        _copy_out(out1)


@functools.lru_cache(maxsize=None)
def _get_sc_agg():
    mesh = plsc.VectorSubcoreMesh(
        core_axis_name="c", subcore_axis_name="s",
        num_cores=2, num_subcores=16)
    return pl.kernel(
        _sc_agg_body,
        out_type=[jax.ShapeDtypeStruct((N, H), _f32),
                  jax.ShapeDtypeStruct((N, H), _f32)],
        mesh=mesh,
        scratch_types=[
            pltpu.VMEM((CH,), jnp.int32),
            pltpu.VMEM((CH,), jnp.int32),
            pltpu.VMEM((CH,), jnp.int32),
            pltpu.VMEM((CH,), jnp.int32),
            pltpu.VMEM((CH, H), _f32),
            pltpu.VMEM((CH, H), _f32),
            pltpu.VMEM_SHARED((ACC_ROWS, H), _f32),
            pltpu.SemaphoreType.DMA,
            pltpu.SemaphoreType.DMA,
            pltpu.SemaphoreType.DMA,
            pltpu.SemaphoreType.DMA,
        ],
    )


def _gelu(y):
    return 0.5 * y * (1.0 + lax.erf(y * 0.7071067811865476))


def _layer_a_body(h0_ref, h1_ref, a0_ref, a1_ref, wr_ref, br_ref, w1_ref,
                  b1_ref, w2_ref, b2_ref, eps_ref, h2_ref, id_ref, st_ref):
    i = pl.program_id(0)
    e1 = 1.0 + eps_ref[0, 0]
    h0 = h0_ref[...]
    h1 = h1_ref[...]
    u0 = e1 * h0 + a0_ref[...]
    u1 = e1 * h1 + a1_ref[...]
    ident = (jnp.dot(h0, wr_ref[0:H, :], preferred_element_type=_f32)
             + jnp.dot(h1, wr_ref[H:D, :], preferred_element_type=_f32)
             + br_ref[...])
    t = (jnp.dot(u0, w1_ref[0:H, :], preferred_element_type=_f32)
         + jnp.dot(u1, w1_ref[H:D, :], preferred_element_type=_f32)
         + b1_ref[...])
    t = jnp.maximum(t, 0.0)
    h2 = jnp.dot(t, w2_ref[...], preferred_element_type=_f32) + b2_ref[...]
    h2_ref[...] = h2
    id_ref[...] = ident

    @pl.when(i == 0)
    def _():
        st_ref[...] = jnp.zeros((2, D), _f32)

    st_ref[0:1, :] += jnp.sum(h2, axis=0, keepdims=True)
    st_ref[1:2, :] += jnp.sum(h2 * h2, axis=0, keepdims=True)


def _layer_a(h0, h1, a0, a1, wr, br, w1, b1, w2, b2, eps):
    blk = lambda w: pl.BlockSpec((R, w), lambda i: (i, 0))
    full = lambda a, b: pl.BlockSpec((a, b), lambda i: (0, 0))
    return pl.pallas_call(
        _layer_a_body,
        grid=(GRID,),
        in_specs=[blk(H), blk(H), blk(H), blk(H),
                  full(D, D), full(1, D), full(D, D), full(1, D),
                  full(D, D), full(1, D), full(1, 1)],
        out_specs=[blk(D), blk(D), full(2, D)],
        out_shape=[jax.ShapeDtypeStruct((N, D), _f32),
                   jax.ShapeDtypeStruct((N, D), _f32),
                   jax.ShapeDtypeStruct((2, D), _f32)],
    )(h0, h1, a0, a1, wr, br, w1, b1, w2, b2, eps)


def _layer_b_body(h2_ref, id_ref, st_ref, g_ref, be_ref, o0_ref, o1_ref):
    st = st_ref[...]
    mu = st[0:1, :] * (1.0 / N)
    var = st[1:2, :] * (1.0 / N) - mu * mu
    inv = lax.rsqrt(var + 1e-5) * g_ref[...]
    y = (h2_ref[...] - mu) * inv + be_ref[...] + id_ref[...]
    y = _gelu(y)
    o0_ref[...] = y[:, 0:H]
    o1_ref[...] = y[:, H:D]


def _layer_b(h2, ident, st, gamma, beta):
    blk = lambda w: pl.BlockSpec((R, w), lambda i: (i, 0))
    full = lambda a, b: pl.BlockSpec((a, b), lambda i: (0, 0))
    return pl.pallas_call(
        _layer_b_body,
        grid=(GRID,),
        in_specs=[blk(D), blk(D), full(2, D), full(1, D), full(1, D)],
        out_specs=[blk(H), blk(H)],
        out_shape=[jax.ShapeDtypeStruct((N, H), _f32),
                   jax.ShapeDtypeStruct((N, H), _f32)],
    )(h2, ident, st, gamma, beta)


def _head_body(h0_ref, h1_ref, b_ref, wf1_ref, bf1_ref, wf2_ref, bf2_ref,
               out_ref, sums, cntm):
    i = pl.program_id(0)

    @pl.when(i == 0)
    def _():
        sums[...] = jnp.zeros((G, D), _f32)
        cntm[...] = jnp.zeros((G, H), _f32)

    oh = (b_ref[...] == lax.broadcasted_iota(jnp.int32, (R, G), 1)).astype(_f32)
    dn = (((0,), (0,)), ((), ()))
    sums[:, 0:H] += lax.dot_general(oh, h0_ref[...], dn,
                                    preferred_element_type=_f32)
    sums[:, H:D] += lax.dot_general(oh, h1_ref[...], dn,
                                    preferred_element_type=_f32)
    cntm[...] += lax.dot_general(oh, jnp.ones((R, H), _f32), dn,
                                 preferred_element_type=_f32)

    @pl.when(i == GRID - 1)
    def _():
        cnt = jnp.maximum(cntm[...], 1.0)
        p0 = sums[:, 0:H] / cnt
        p1 = sums[:, H:D] / cnt
        z = (jnp.dot(p0, wf1_ref[0:H, :], preferred_element_type=_f32)
             + jnp.dot(p1, wf1_ref[H:D, :], preferred_element_type=_f32)
             + bf1_ref[...])
        z = _gelu(z)
        out_ref[...] = (jnp.dot(z, wf2_ref[...], preferred_element_type=_f32)
                        + bf2_ref[...])


def _head(h0, h1, batch2, wf1, bf1, wf2, bf2):
    blk = lambda w: pl.BlockSpec((R, w), lambda i: (i, 0))
    full = lambda a, b: pl.BlockSpec((a, b), lambda i: (0, 0))
    return pl.pallas_call(
        _head_body,
        grid=(GRID,),
        in_specs=[blk(H), blk(H), blk(1),
                  full(D, D), full(1, D), full(D, 10), full(1, 10)],
        out_specs=pl.BlockSpec((G, 10), lambda i: (0, 0)),
        out_shape=jax.ShapeDtypeStruct((G, 10), _f32),
        scratch_shapes=[pltpu.VMEM((G, D), _f32), pltpu.VMEM((G, H), _f32)],
    )(h0, h1, batch2, wf1, bf1, wf2, bf2)


def kernel(x, edge_index, batch, params):
    src = edge_index[0]
    dst = edge_index[1]
    npad = E_PAD - E
    ar = jnp.arange(npad, dtype=jnp.int32)
    srcp = jnp.concatenate([src, (ar * 997) % N])
    dstp = jnp.concatenate([dst, N + (ar % 16)])
    batch2 = batch.reshape(N, 1)

    h0 = x[:, 0:H]
    h1 = x[:, H:D]
    for l in range(4):
        g = params[f"gin{l}"]
        bn = params[f"bn{l}"]
        rs = params[f"res{l}"]
        a0, a1 = _get_sc_agg()(h0, h1, srcp, dstp)
        h2, ident, st = _layer_a(
            h0, h1, a0, a1, rs["W"], rs["b"].reshape(1, D),
            g["lin1"]["W"], g["lin1"]["b"].reshape(1, D),
            g["lin2"]["W"], g["lin2"]["b"].reshape(1, D),
            g["eps"].reshape(1, 1))
        h0, h1 = _layer_b(h2, ident, st, bn["gamma"].reshape(1, D),
                          bn["beta"].reshape(1, D))
    return _head(h0, h1, batch2, params["fc1"]["W"],
                 params["fc1"]["b"].reshape(1, D), params["fc2"]["W"],
                 params["fc2"]["b"].reshape(1, 10))
